# baseline (speedup 1.0000x reference)
import jax, jax.numpy as jnp
from jax.experimental import pallas as pl
from jax.experimental.pallas import tpu as pltpu

def _k(a_ref, b_ref, c_ref, d_ref, e_ref, out_ref):
    s = (jnp.sum(a_ref[...] * a_ref[...]) + jnp.sum(b_ref[...] * b_ref[...])
         + jnp.sum(c_ref[...] * c_ref[...]) + jnp.sum(d_ref[...] * d_ref[...])
         + jnp.sum(e_ref[...] * e_ref[...]))
    lane = jax.lax.broadcasted_iota(jnp.int32, (1, 1, 128), 2)
    out_ref[...] = jnp.where(lane == 0, s, 0.0)

def kernel(pred_node, gt_node, pred_line, gt_line, adj, line_param, node_count):
    pn = pred_node.reshape(16, 400, 128)
    gn = gt_node.reshape(16, 400, 128)
    pline = pred_line.reshape(16, 392, 128)
    gline = gt_line.reshape(16, 392, 128)
    lpar = line_param.reshape(16, 392, 128)
    spec_n = pl.BlockSpec((1, 400, 128), lambda i: (i, 0, 0))
    spec_l = pl.BlockSpec((1, 392, 128), lambda i: (i, 0, 0))
    sums = pl.pallas_call(
        _k,
        grid=(16,),
        in_specs=[spec_n, spec_n, spec_l, spec_l, spec_l],
        out_specs=pl.BlockSpec((1, 1, 128), lambda i: (i, 0, 0)),
        out_shape=jax.ShapeDtypeStruct((16, 1, 128), jnp.float32),
    )(pn, gn, pline, gline, lpar)
    s = jnp.sum(sums[:, 0, 0])
    return (s, s, s)


# speedup vs baseline: 41.7680x; 41.7680x over previous
import jax
import jax.numpy as jnp
from jax.experimental import pallas as pl
from jax.experimental.pallas import tpu as pltpu

B = 4096
N = 50
L = N - 1
LAMBDA = 0.5

RB = 32
ROWS = B // 128 * 4
GRID = ROWS // RB


def _loss_kernel(pn_ref, gn_ref, pl_ref, gl_ref, lp_ref, out_ref):
    x = pn_ref[...]
    g = gn_ref[...]
    y = pl_ref[...]
    gl = gl_ref[...]
    z = lp_ref[...]

    dn = x - g
    s1 = jnp.sum(dn * dn)
    dl = y - gl
    s2 = jnp.sum(dl * dl)

    ys = jnp.concatenate([y[:, 2:, :], jnp.zeros((L, 2, 128), jnp.float32)],
                         axis=1)
    ysn = jnp.concatenate([ys[1:L], jnp.zeros((1, RB, 128), jnp.float32)],
                          axis=0)
    err = ys - ysn - x[1:N]
    riota = jax.lax.broadcasted_iota(jnp.int32, (1, RB, 1), 1)
    s3 = jnp.sum(jnp.where(riota % 4 < 2, err * err, 0.0))

    u = z * ys
    gsum = u + jnp.concatenate([u[:, 1:, :],
                                jnp.zeros((L, 1, 128), jnp.float32)], axis=1)
    xs2 = jnp.concatenate([x[:, 2:, :], jnp.zeros((N, 2, 128), jnp.float32)],
                          axis=1)
    v2 = xs2 * xs2
    dv2 = v2[0:L] - v2[1:N]
    lf = 2.0 * gsum - dv2
    s4 = jnp.sum(jnp.where(riota % 4 == 0, lf * lf, 0.0))

    lane = jax.lax.broadcasted_iota(jnp.int32, (1, 1, 128), 2)
    packed = (jnp.where(lane == 0, s1, 0.0) + jnp.where(lane == 1, s2, 0.0)
              + jnp.where(lane == 2, s3, 0.0) + jnp.where(lane == 3, s4, 0.0))

    @pl.when(pl.program_id(0) == 0)
    def _init():
        out_ref[...] = packed

    @pl.when(pl.program_id(0) != 0)
    def _acc():
        out_ref[...] = out_ref[...] + packed


def _to_t(x, n):
    return (x.transpose(1, 2, 0).reshape(n, 4, B // 128, 128)
            .transpose(0, 2, 1, 3).reshape(n, B // 128 * 4, 128))


def kernel(pred_node, gt_node, pred_line, gt_line, adj, line_param, node_count):
    del adj, node_count
    pn = _to_t(pred_node, N)
    gn = _to_t(gt_node, N)
    pline = _to_t(pred_line, L)
    gline = _to_t(gt_line, L)
    lpar = _to_t(line_param, L)

    spec_n = pl.BlockSpec((N, RB, 128), lambda i: (0, i, 0))
    spec_l = pl.BlockSpec((L, RB, 128), lambda i: (0, i, 0))

    sums = pl.pallas_call(
        _loss_kernel,
        grid=(GRID,),
        in_specs=[spec_n, spec_n, spec_l, spec_l, spec_l],
        out_specs=pl.BlockSpec((1, 1, 128), lambda i: (0, 0, 0)),
        out_shape=jax.ShapeDtypeStruct((1, 1, 128), jnp.float32),
    )(pn, gn, pline, gline, lpar)

    s1 = sums[0, 0, 0]
    s2 = sums[0, 0, 1]
    s3 = sums[0, 0, 2]
    s4 = sums[0, 0, 3]

    node_mse = s1 / (B * N * 4)
    line_mse = s2 / (B * L * 4)
    pred_loss = node_mse + line_mse
    physics_loss = s3 / (B * N * 2) + s4 / (B * L)
    total_loss = pred_loss + LAMBDA * physics_loss
    return (total_loss, pred_loss, physics_loss)


# speedup vs baseline: 43.1934x; 1.0341x over previous
import jax
import jax.numpy as jnp
from jax.experimental import pallas as pl
from jax.experimental.pallas import tpu as pltpu

B = 4096
N = 50
L = N - 1
LAMBDA = 0.5

RB = 64
ROWS = B // 128 * 4
GRID = ROWS // RB


def _loss_kernel(pn_ref, gn_ref, pl_ref, gl_ref, lp_ref, out_ref):
    x = pn_ref[...]
    g = gn_ref[...]
    y = pl_ref[...]
    gl = gl_ref[...]
    z = lp_ref[...]

    dn = x - g
    s1 = jnp.sum(dn * dn)
    dl = y - gl
    s2 = jnp.sum(dl * dl)

    ys = jnp.concatenate([y[:, 2:, :], jnp.zeros((L, 2, 128), jnp.float32)],
                         axis=1)
    ysn = jnp.concatenate([ys[1:L], jnp.zeros((1, RB, 128), jnp.float32)],
                          axis=0)
    err = ys - ysn - x[1:N]
    riota = jax.lax.broadcasted_iota(jnp.int32, (1, RB, 1), 1)
    s3 = jnp.sum(jnp.where(riota % 4 < 2, err * err, 0.0))

    u = z * ys
    gsum = u + jnp.concatenate([u[:, 1:, :],
                                jnp.zeros((L, 1, 128), jnp.float32)], axis=1)
    xs2 = jnp.concatenate([x[:, 2:, :], jnp.zeros((N, 2, 128), jnp.float32)],
                          axis=1)
    v2 = xs2 * xs2
    dv2 = v2[0:L] - v2[1:N]
    lf = 2.0 * gsum - dv2
    s4 = jnp.sum(jnp.where(riota % 4 == 0, lf * lf, 0.0))

    lane = jax.lax.broadcasted_iota(jnp.int32, (1, 1, 128), 2)
    packed = (jnp.where(lane == 0, s1, 0.0) + jnp.where(lane == 1, s2, 0.0)
              + jnp.where(lane == 2, s3, 0.0) + jnp.where(lane == 3, s4, 0.0))

    @pl.when(pl.program_id(0) == 0)
    def _init():
        out_ref[...] = packed

    @pl.when(pl.program_id(0) != 0)
    def _acc():
        out_ref[...] = out_ref[...] + packed


def _to_t(x, n):
    return (x.transpose(1, 2, 0).reshape(n, 4, B // 128, 128)
            .transpose(0, 2, 1, 3).reshape(n, B // 128 * 4, 128))


def kernel(pred_node, gt_node, pred_line, gt_line, adj, line_param, node_count):
    del adj, node_count
    pn = _to_t(pred_node, N)
    gn = _to_t(gt_node, N)
    pline = _to_t(pred_line, L)
    gline = _to_t(gt_line, L)
    lpar = _to_t(line_param, L)

    spec_n = pl.BlockSpec((N, RB, 128), lambda i: (0, i, 0))
    spec_l = pl.BlockSpec((L, RB, 128), lambda i: (0, i, 0))

    sums = pl.pallas_call(
        _loss_kernel,
        grid=(GRID,),
        in_specs=[spec_n, spec_n, spec_l, spec_l, spec_l],
        out_specs=pl.BlockSpec((1, 1, 128), lambda i: (0, 0, 0)),
        out_shape=jax.ShapeDtypeStruct((1, 1, 128), jnp.float32),
    )(pn, gn, pline, gline, lpar)

    s1 = sums[0, 0, 0]
    s2 = sums[0, 0, 1]
    s3 = sums[0, 0, 2]
    s4 = sums[0, 0, 3]

    node_mse = s1 / (B * N * 4)
    line_mse = s2 / (B * L * 4)
    pred_loss = node_mse + line_mse
    physics_loss = s3 / (B * N * 2) + s4 / (B * L)
    total_loss = pred_loss + LAMBDA * physics_loss
    return (total_loss, pred_loss, physics_loss)
